# Initial kernel scaffold; baseline (speedup 1.0000x reference)
import jax
import jax.numpy as jnp
from jax.experimental import pallas as pl


def kernel(input_ids, token_type_ids, position_ids, random_walk, anonymous_random_walk, word_emb, pos_emb, type_emb, bias_emb, ln_gamma, ln_beta):
    raise NotImplementedError("write your pallas kernel here")


# speedup vs baseline: 5.0227x; 5.0227x over previous
import functools

import jax
import jax.numpy as jnp
from jax import lax
from jax.experimental import pallas as pl
from jax.experimental.pallas import tpu as pltpu
from jax.experimental.pallas import tpu_sc as plsc

B, S, H = 1024, 20, 64
NT = B * S
EPS = 1e-12

NC, NS = 2, 16
NWORK = NC * NS
TPW = NT // NWORK
CHUNK = 128
NBUF = 5
RW_CHUNKS = TPW * 32 // CHUNK
TOK_CHUNKS = TPW // CHUNK
SC_TOKENS = NS * TPW
ACC_ROWS = SC_TOKENS + 8
TRASH = SC_TOKENS


def _tile_body(bias_hbm, word_hbm, pos_hbm, type_hbm,
               rw_hbm, arw_hbm, wi_hbm, pi_hbm, ti_hbm,
               out_hbm,
               rw_v, arw_v, wpt_v, rows_v, dsti_v, acc_sh,
               g0, g1, g2, g3, g4, s0, s1, s2, s3, s4):
    gsems = (g0, g1, g2, g3, g4)
    ssems = (s0, s1, s2, s3, s4)
    c = lax.axis_index("c")
    s = lax.axis_index("s")
    wid = c * NS + s
    sbase = s * TPW
    gbase = wid * TPW
    trash_vec = jnp.zeros((16,), jnp.int32) + TRASH

    pltpu.sync_copy(rw_hbm.at[pl.ds(wid * RW_CHUNKS, RW_CHUNKS)], rw_v)
    pltpu.sync_copy(arw_hbm.at[pl.ds(wid * RW_CHUNKS, RW_CHUNKS)], arw_v)
    pltpu.sync_copy(wi_hbm.at[pl.ds(wid * 8, 8)], wpt_v.at[pl.ds(0, 8)])
    pltpu.sync_copy(pi_hbm.at[pl.ds(wid * 8, 8)], wpt_v.at[pl.ds(8, 8)])
    pltpu.sync_copy(ti_hbm.at[pl.ds(wid * 8, 8)], wpt_v.at[pl.ds(16, 8)])

    def build_token_dst(b, j, idx_row, masked):
        for v in range(8):
            dvec = (jnp.zeros((16,), jnp.int32)
                    + (sbase + j * CHUNK + 16 * v)
                    + lax.iota(jnp.int32, 16))
            if masked:
                ivec = wpt_v[idx_row, pl.ds(16 * v, 16)]
                dvec = jnp.where(ivec == 0, trash_vec, dvec)
            dsti_v[b, pl.ds(16 * v, 16)] = dvec

    def scatter_wait(b):
        pltpu.make_async_copy(rows_v.at[b], acc_sh.at[dsti_v.at[b]],
                              ssems[b]).wait()

    def drain():
        for b in range(NBUF):
            scatter_wait(b)

    for b in range(NBUF):
        row = 8 + b
        gd = pltpu.async_copy(pos_hbm.at[wpt_v.at[row]], rows_v.at[b],
                              gsems[b])
        build_token_dst(b, b, row, masked=False)
        gd.wait()
        pltpu.async_copy(rows_v.at[b], acc_sh.at[dsti_v.at[b]], ssems[b])
    drain()

    for b in range(NBUF):
        gd = pltpu.async_copy(word_hbm.at[wpt_v.at[b]], rows_v.at[b],
                              gsems[b])
        build_token_dst(b, b, b, masked=True)
        gd.wait()
        pltpu.async_copy(rows_v.at[b], acc_sh.at[dsti_v.at[b]], ssems[b],
                         add=True)
    drain()

    for b in range(NBUF):
        row = 16 + b
        gd = pltpu.async_copy(type_hbm.at[wpt_v.at[row]], rows_v.at[b],
                              gsems[b])
        build_token_dst(b, b, row, masked=False)
        gd.wait()
        pltpu.async_copy(rows_v.at[b], acc_sh.at[dsti_v.at[b]], ssems[b],
                         add=True)
    drain()

    def walk_pass(idx_v):
        def outer(j, carry):
            for b in range(NBUF):
                ch = j * NBUF + b

                @pl.when(j >= 1)
                def _():
                    scatter_wait(b)

                gd = pltpu.async_copy(bias_hbm.at[idx_v.at[ch]],
                                      rows_v.at[b], gsems[b])
                for v in range(8):
                    ivec = idx_v[ch, pl.ds(16 * v, 16)]
                    dvec = (jnp.zeros((16,), jnp.int32)
                            + (sbase + ch * 4 + v // 2))
                    dvec = jnp.where(ivec == 0, trash_vec, dvec)
                    dsti_v[b, pl.ds(16 * v, 16)] = dvec
                gd.wait()
                pltpu.async_copy(rows_v.at[b], acc_sh.at[dsti_v.at[b]],
                                 ssems[b], add=True)
            return carry

        lax.fori_loop(0, RW_CHUNKS // NBUF, outer, 0)
        drain()

    walk_pass(rw_v)
    walk_pass(arw_v)

    for jb in range(TOK_CHUNKS):
        pltpu.sync_copy(acc_sh.at[pl.ds(sbase + jb * CHUNK, CHUNK)],
                        rows_v.at[0])
        pltpu.sync_copy(rows_v.at[0],
                        out_hbm.at[pl.ds(gbase + jb * CHUNK, CHUNK)])


@jax.jit
def _sge_sums(bias_emb, word_emb, pos_emb, type_emb, rw2, arw2, wi2, pi2, ti2):
    k = pl.kernel(
        _tile_body,
        out_type=jax.ShapeDtypeStruct((NT, H), jnp.float32),
        mesh=plsc.VectorSubcoreMesh(core_axis_name="c", subcore_axis_name="s"),
        compiler_params=pltpu.CompilerParams(use_tc_tiling_on_sc=False),
        scratch_types=[
            pltpu.VMEM((RW_CHUNKS, CHUNK), jnp.int32),
            pltpu.VMEM((RW_CHUNKS, CHUNK), jnp.int32),
            pltpu.VMEM((24, CHUNK), jnp.int32),
            pltpu.VMEM((NBUF, CHUNK, H), jnp.float32),
            pltpu.VMEM((NBUF, CHUNK), jnp.int32),
            pltpu.VMEM_SHARED((ACC_ROWS, H), jnp.float32),
        ] + [pltpu.SemaphoreType.DMA] * (2 * NBUF),
    )
    return k(bias_emb, word_emb, pos_emb, type_emb, rw2, arw2, wi2, pi2, ti2)


def _ln_body(x_ref, g_ref, b_ref, o_ref):
    x = x_ref[...]
    mu = jnp.mean(x, axis=-1, keepdims=True)
    xc = x - mu
    var = jnp.mean(xc * xc, axis=-1, keepdims=True)
    o_ref[...] = xc * lax.rsqrt(var + EPS) * g_ref[...] + b_ref[...]


@jax.jit
def _layer_norm(x, gamma, beta):
    rows = 1024
    return pl.pallas_call(
        _ln_body,
        grid=(NT // rows,),
        in_specs=[
            pl.BlockSpec((rows, H), lambda i: (i, 0)),
            pl.BlockSpec((1, H), lambda i: (0, 0)),
            pl.BlockSpec((1, H), lambda i: (0, 0)),
        ],
        out_specs=pl.BlockSpec((rows, H), lambda i: (i, 0)),
        out_shape=jax.ShapeDtypeStruct((NT, H), jnp.float32),
    )(x, gamma, beta)


def _pad8(x):
    x3 = x.astype(jnp.int32).reshape(NWORK, TOK_CHUNKS, CHUNK)
    x3 = jnp.pad(x3, ((0, 0), (0, 8 - TOK_CHUNKS), (0, 0)))
    return x3.reshape(NWORK * 8, CHUNK)


def kernel(input_ids, token_type_ids, position_ids, random_walk,
           anonymous_random_walk, word_emb, pos_emb, type_emb, bias_emb,
           ln_gamma, ln_beta):
    wi2 = _pad8(input_ids.reshape(NT))
    pi2 = _pad8(position_ids.reshape(NT))
    ti2 = _pad8(token_type_ids.reshape(NT))
    rw2 = random_walk.astype(jnp.int32).reshape(NWORK * RW_CHUNKS, CHUNK)
    arw2 = anonymous_random_walk.astype(jnp.int32).reshape(
        NWORK * RW_CHUNKS, CHUNK)
    sums = _sge_sums(bias_emb, word_emb, pos_emb, type_emb,
                     rw2, arw2, wi2, pi2, ti2)
    out = _layer_norm(sums, ln_gamma.reshape(1, H), ln_beta.reshape(1, H))
    return out.reshape(B, S, H)


# speedup vs baseline: 5.5641x; 1.1078x over previous
import functools

import jax
import jax.numpy as jnp
from jax import lax
from jax.experimental import pallas as pl
from jax.experimental.pallas import tpu as pltpu
from jax.experimental.pallas import tpu_sc as plsc

B, S, H = 1024, 20, 64
NT = B * S
EPS = 1e-12

NC, NS = 2, 16
NWORK = NC * NS
TPW = NT // NWORK
CHUNK = 128
NBUF = 5
WALK_CHUNKS = 2 * TPW * 32 // CHUNK
TOK_CHUNKS = TPW // CHUNK
SC_TOKENS = NS * TPW
ACC_ROWS = SC_TOKENS + 8
TRASH = SC_TOKENS


def _tile_body(bias_hbm, word_hbm, pos_hbm, type_hbm,
               walk_hbm, wi_hbm, pi_hbm, ti_hbm,
               out_hbm,
               walk_v, wpt_v, rows_v, dsti_v, acc_sh,
               g0, g1, g2, g3, g4, s0, s1, s2, s3, s4):
    gsems = (g0, g1, g2, g3, g4)
    ssems = (s0, s1, s2, s3, s4)
    c = lax.axis_index("c")
    s = lax.axis_index("s")
    wid = c * NS + s
    sbase = s * TPW
    gbase = wid * TPW
    trash_vec = jnp.zeros((16,), jnp.int32) + TRASH

    pltpu.sync_copy(walk_hbm.at[pl.ds(wid * WALK_CHUNKS, WALK_CHUNKS)],
                    walk_v)
    pltpu.sync_copy(wi_hbm.at[pl.ds(wid * 8, 8)], wpt_v.at[pl.ds(0, 8)])
    pltpu.sync_copy(pi_hbm.at[pl.ds(wid * 8, 8)], wpt_v.at[pl.ds(8, 8)])
    pltpu.sync_copy(ti_hbm.at[pl.ds(wid * 8, 8)], wpt_v.at[pl.ds(16, 8)])

    def build_token_dst(b, idx_row, masked):
        for v in range(8):
            dvec = (jnp.zeros((16,), jnp.int32)
                    + (sbase + b * CHUNK + 16 * v)
                    + lax.iota(jnp.int32, 16))
            if masked:
                ivec = wpt_v[idx_row, pl.ds(16 * v, 16)]
                dvec = jnp.where(ivec == 0, trash_vec, dvec)
            dsti_v[b, pl.ds(16 * v, 16)] = dvec

    def scatter_wait(b):
        pltpu.make_async_copy(rows_v.at[b], acc_sh.at[dsti_v.at[b]],
                              ssems[b]).wait()

    def gather_wait(b):
        pltpu.make_async_copy(bias_hbm.at[dsti_v.at[b]], rows_v.at[b],
                              gsems[b]).wait()

    for b in range(NBUF):
        pltpu.async_copy(pos_hbm.at[wpt_v.at[8 + b]], rows_v.at[b], gsems[b])
    for b in range(NBUF):
        gather_wait(b)
        build_token_dst(b, 8 + b, masked=False)
        pltpu.async_copy(rows_v.at[b], acc_sh.at[dsti_v.at[b]], ssems[b])
    for b in range(NBUF):
        scatter_wait(b)

    for b in range(NBUF):
        pltpu.async_copy(word_hbm.at[wpt_v.at[b]], rows_v.at[b], gsems[b])
    for b in range(NBUF):
        gather_wait(b)
        build_token_dst(b, b, masked=True)
        pltpu.async_copy(rows_v.at[b], acc_sh.at[dsti_v.at[b]], ssems[b],
                         add=True)

    for b in range(NBUF):
        scatter_wait(b)
        pltpu.async_copy(type_hbm.at[wpt_v.at[16 + b]], rows_v.at[b],
                         gsems[b])
    for b in range(NBUF):
        gather_wait(b)
        build_token_dst(b, 16 + b, masked=False)
        pltpu.async_copy(rows_v.at[b], acc_sh.at[dsti_v.at[b]], ssems[b],
                         add=True)

    def outer(j, carry):
        for b in range(NBUF):
            scatter_wait(b)
            ch = j * NBUF + b
            pltpu.async_copy(bias_hbm.at[walk_v.at[ch]], rows_v.at[b],
                             gsems[b])
        for b in range(NBUF):
            ch = j * NBUF + b
            tok0 = sbase + lax.rem(ch, WALK_CHUNKS // 2) * 4
            gather_wait(b)
            for v in range(8):
                ivec = walk_v[ch, pl.ds(16 * v, 16)]
                dvec = jnp.zeros((16,), jnp.int32) + (tok0 + v // 2)
                dvec = jnp.where(ivec == 0, trash_vec, dvec)
                dsti_v[b, pl.ds(16 * v, 16)] = dvec
            pltpu.async_copy(rows_v.at[b], acc_sh.at[dsti_v.at[b]],
                             ssems[b], add=True)
        return carry

    lax.fori_loop(0, WALK_CHUNKS // NBUF, outer, 0)
    for b in range(NBUF):
        scatter_wait(b)

    for jb in range(TOK_CHUNKS):
        pltpu.async_copy(acc_sh.at[pl.ds(sbase + jb * CHUNK, CHUNK)],
                         rows_v.at[jb], gsems[jb])
    for jb in range(TOK_CHUNKS):
        pltpu.make_async_copy(acc_sh.at[pl.ds(sbase + jb * CHUNK, CHUNK)],
                              rows_v.at[jb], gsems[jb]).wait()
        pltpu.async_copy(rows_v.at[jb],
                         out_hbm.at[pl.ds(gbase + jb * CHUNK, CHUNK)],
                         ssems[jb])
    for jb in range(TOK_CHUNKS):
        pltpu.make_async_copy(rows_v.at[jb],
                              out_hbm.at[pl.ds(gbase + jb * CHUNK, CHUNK)],
                              ssems[jb]).wait()


@jax.jit
def _sge_sums(bias_emb, word_emb, pos_emb, type_emb, walk2, wi2, pi2, ti2):
    k = pl.kernel(
        _tile_body,
        out_type=jax.ShapeDtypeStruct((NT, H), jnp.float32),
        mesh=plsc.VectorSubcoreMesh(core_axis_name="c", subcore_axis_name="s"),
        compiler_params=pltpu.CompilerParams(use_tc_tiling_on_sc=False),
        scratch_types=[
            pltpu.VMEM((WALK_CHUNKS, CHUNK), jnp.int32),
            pltpu.VMEM((24, CHUNK), jnp.int32),
            pltpu.VMEM((NBUF, CHUNK, H), jnp.float32),
            pltpu.VMEM((NBUF, CHUNK), jnp.int32),
            pltpu.VMEM_SHARED((ACC_ROWS, H), jnp.float32),
        ] + [pltpu.SemaphoreType.DMA] * (2 * NBUF),
    )
    return k(bias_emb, word_emb, pos_emb, type_emb, walk2, wi2, pi2, ti2)


def _ln_body(x_ref, g_ref, b_ref, o_ref):
    x = x_ref[...]
    mu = jnp.mean(x, axis=-1, keepdims=True)
    xc = x - mu
    var = jnp.mean(xc * xc, axis=-1, keepdims=True)
    o_ref[...] = xc * lax.rsqrt(var + EPS) * g_ref[...] + b_ref[...]


@jax.jit
def _layer_norm(x, gamma, beta):
    rows = 1024
    return pl.pallas_call(
        _ln_body,
        grid=(NT // rows,),
        in_specs=[
            pl.BlockSpec((rows, H), lambda i: (i, 0)),
            pl.BlockSpec((1, H), lambda i: (0, 0)),
            pl.BlockSpec((1, H), lambda i: (0, 0)),
        ],
        out_specs=pl.BlockSpec((rows, H), lambda i: (i, 0)),
        out_shape=jax.ShapeDtypeStruct((NT, H), jnp.float32),
    )(x, gamma, beta)


def _pad8(x):
    x3 = x.astype(jnp.int32).reshape(NWORK, TOK_CHUNKS, CHUNK)
    x3 = jnp.pad(x3, ((0, 0), (0, 8 - TOK_CHUNKS), (0, 0)))
    return x3.reshape(NWORK * 8, CHUNK)


def kernel(input_ids, token_type_ids, position_ids, random_walk,
           anonymous_random_walk, word_emb, pos_emb, type_emb, bias_emb,
           ln_gamma, ln_beta):
    wi2 = _pad8(input_ids.reshape(NT))
    pi2 = _pad8(position_ids.reshape(NT))
    ti2 = _pad8(token_type_ids.reshape(NT))
    rw3 = random_walk.astype(jnp.int32).reshape(NWORK, WALK_CHUNKS // 2,
                                                CHUNK)
    arw3 = anonymous_random_walk.astype(jnp.int32).reshape(
        NWORK, WALK_CHUNKS // 2, CHUNK)
    walk2 = jnp.concatenate([rw3, arw3], axis=1).reshape(
        NWORK * WALK_CHUNKS, CHUNK)
    sums = _sge_sums(bias_emb, word_emb, pos_emb, type_emb,
                     walk2, wi2, pi2, ti2)
    out = _layer_norm(sums, ln_gamma.reshape(1, H), ln_beta.reshape(1, H))
    return out.reshape(B, S, H)
